# baseline (speedup 1.0000x reference)
import functools

import jax
import jax.numpy as jnp
from jax import lax
from jax.experimental import pallas as pl
from jax.experimental.pallas import tpu as pltpu
from jax.experimental.pallas import tpu_sc as plsc

F = 32
R = 512
HW = R * R
N = 524288
NW = 32
PTS = N // NW
B = 64
NCH = PTS // B
NPAIR = NCH // 2
G = B // 16
NROW = 6 * 128
OUTW = 3 * F


def _axis_decompose(v):
    t = (v - 0.0) / 2.0 + 0.5
    xn = t * 2.0 - 1.0
    pos = (xn + 1.0) * 0.5 * float(R - 1)
    it = pos.astype(jnp.int32)
    itf = it.astype(jnp.float32)
    fl = jnp.where(pos < itf, it - 1, it)
    w = pos - fl.astype(jnp.float32)
    i0 = jnp.clip(fl, 0, R - 1)
    i1 = jnp.clip(fl + 1, 0, R - 1)
    return i0, i1, w


def _body(xs, ys, zs, t0, t1, t2, out,
          xall, yall, zall, wA, wB, idxA, idxB, rowsA, rowsB, outA, outB,
          gsemA, gsemB, osemA, osemB):
    wid = lax.axis_index("s") * 2 + lax.axis_index("c")
    tbase = wid * PTS
    iota2 = lax.iota(jnp.int32, 16) * 2
    tbls = (t0, t0, t1, t1, t2, t2)

    pltpu.sync_copy(xs.at[pl.ds(tbase, PTS)], xall)
    pltpu.sync_copy(ys.at[pl.ds(tbase, PTS)], yall)
    pltpu.sync_copy(zs.at[pl.ds(tbase, PTS)], zall)

    def build(kc, idx_t, w_t):
        cb = kc * B

        def grp(g, c2):
            col = g * 16
            sl = pl.ds(cb + col, 16)
            x0, x1, wx = _axis_decompose(xall[sl])
            y0, y1, wy = _axis_decompose(yall[sl])
            z0, z1, wz = _axis_decompose(zall[sl])
            w_t[pl.ds(col, 16)] = wx
            w_t[pl.ds(B + col, 16)] = wy
            w_t[pl.ds(2 * B + col, 16)] = wz
            planes = ((x0, x1, y0, y1), (x0, x1, z0, z1), (y0, y1, z0, z1))
            for p, (a0, a1, b0, b1) in enumerate(planes):
                rb0 = b0 * R
                rb1 = b1 * R
                for c, rr in enumerate((rb0 + a0, rb0 + a1, rb1 + a0, rb1 + a1)):
                    q = 4 * p + c
                    idx_t[q // 2, pl.ds((q % 2) * 64 + col, 16)] = rr
            return c2

        lax.fori_loop(0, G, grp, 0)

    def fire_gather(idx_t, rows_t, sem):
        for j in range(6):
            pltpu.make_async_copy(tbls[j].at[idx_t.at[j]],
                                  rows_t.at[pl.ds(j * 128, 128)], sem).start()

    def wait_gather(idx_t, rows_t, sem):
        for j in range(6):
            pltpu.make_async_copy(tbls[j].at[idx_t.at[j]],
                                  rows_t.at[pl.ds(j * 128, 128)], sem).wait()

    def combine(rows_t, w_t, out_t):
        def cgrp(g, c2):
            col = g * 16
            wxg = w_t[pl.ds(col, 16)]
            wyg = w_t[pl.ds(B + col, 16)]
            wzg = w_t[pl.ds(2 * B + col, 16)]
            for j in range(16):
                i = col + j
                jv = jnp.full((16,), j, jnp.int32)
                fmt = plsc.PackFormat.INTERLEAVED
                wxv = wxg.at[jv].get(mode="promise_in_bounds")
                wyv = wyg.at[jv].get(mode="promise_in_bounds")
                wzv = wzg.at[jv].get(mode="promise_in_bounds")
                wxb = plsc.pack(wxv, wxv, format=fmt)
                wyb = plsc.pack(wyv, wyv, format=fmt)
                wzb = plsc.pack(wzv, wzv, format=fmt)
                obase = i * OUTW
                pw = ((wxb, wyb), (wxb, wzb), (wyb, wzb))
                for p, (wa, wb) in enumerate(pw):
                    v00 = rows_t[(4 * p + 0) * 64 + i, :]
                    v01 = rows_t[(4 * p + 1) * 64 + i, :]
                    v10 = rows_t[(4 * p + 2) * 64 + i, :]
                    v11 = rows_t[(4 * p + 3) * 64 + i, :]
                    top = v00 + wa * (v01 - v00)
                    bot = v10 + wa * (v11 - v10)
                    res = top + wb * (bot - top)
                    ue, uo = plsc.unpack(res, format=fmt)
                    plsc.store_scatter(out_t, [iota2 + (obase + p * F + 0)], ue)
                    plsc.store_scatter(out_t, [iota2 + (obase + p * F + 1)], uo)
            return c2

        lax.fori_loop(0, G, cgrp, 0)

    def out_desc(kc, out_t, sem):
        off = (tbase + kc * B) * OUTW
        return pltpu.make_async_copy(out_t, out.at[pl.ds(off, B * OUTW)], sem)

    build(0, idxA, wA)
    fire_gather(idxA, rowsA, gsemA)

    def pair(j, carry):
        k0 = j * 2

        build(k0 + 1, idxB, wB)
        fire_gather(idxB, rowsB, gsemB)
        wait_gather(idxA, rowsA, gsemA)

        @pl.when(j > 0)
        def _():
            out_desc(k0 - 2, outA, osemA).wait()

        combine(rowsA, wA, outA)
        out_desc(k0, outA, osemA).start()

        @pl.when(j < NPAIR - 1)
        def _():
            build(k0 + 2, idxA, wA)
            fire_gather(idxA, rowsA, gsemA)

        wait_gather(idxB, rowsB, gsemB)

        @pl.when(j > 0)
        def _():
            out_desc(k0 - 1, outB, osemB).wait()

        combine(rowsB, wB, outB)
        out_desc(k0 + 1, outB, osemB).start()
        return carry

    lax.fori_loop(0, NPAIR, pair, 0)

    out_desc(NCH - 2, outA, osemA).wait()
    out_desc(NCH - 1, outB, osemB).wait()


_tri = pl.kernel(
    _body,
    out_type=jax.ShapeDtypeStruct((N * OUTW,), jnp.float32),
    mesh=plsc.VectorSubcoreMesh(core_axis_name="c", subcore_axis_name="s"),
    compiler_params=pltpu.CompilerParams(use_tc_tiling_on_sc=False,
                                         needs_layout_passes=False,
                                         disable_bounds_checks=True,
                                         disable_semaphore_checks=True),
    scratch_types=[
        pltpu.VMEM((PTS,), jnp.float32),
        pltpu.VMEM((PTS,), jnp.float32),
        pltpu.VMEM((PTS,), jnp.float32),
        pltpu.VMEM((3 * B,), jnp.float32),
        pltpu.VMEM((3 * B,), jnp.float32),
        pltpu.VMEM((6, 128), jnp.int32),
        pltpu.VMEM((6, 128), jnp.int32),
        pltpu.VMEM((NROW, F), jnp.bfloat16),
        pltpu.VMEM((NROW, F), jnp.bfloat16),
        pltpu.VMEM((B * OUTW,), jnp.float32),
        pltpu.VMEM((B * OUTW,), jnp.float32),
        pltpu.SemaphoreType.DMA,
        pltpu.SemaphoreType.DMA,
        pltpu.SemaphoreType.DMA,
        pltpu.SemaphoreType.DMA,
    ],
)


HB = 8


def _tp_body(pin, pout):
    blk = pin[...]
    pout[...] = (jnp.transpose(blk, (1, 2, 0))
                 .reshape(HB * R, F).astype(jnp.bfloat16))


_tp = pl.pallas_call(
    _tp_body,
    grid=(R // HB,),
    in_specs=[pl.BlockSpec((F, HB, R), lambda i: (0, i, 0))],
    out_specs=pl.BlockSpec((HB * R, F), lambda i: (i, 0)),
    out_shape=jax.ShapeDtypeStruct((HW, F), jnp.bfloat16),
)


def _hwc_table(plane):
    return _tp(plane[0])


@jax.jit
def kernel(x, plane_xy, plane_xz, plane_yz):
    xt = x.T
    flat = _tri(xt[0], xt[1], xt[2],
                _hwc_table(plane_xy), _hwc_table(plane_xz), _hwc_table(plane_yz))
    return flat.reshape(N, OUTW)


# speedup vs baseline: 1.5340x; 1.5340x over previous
import functools

import jax
import jax.numpy as jnp
from jax import lax
from jax.experimental import pallas as pl
from jax.experimental.pallas import tpu as pltpu
from jax.experimental.pallas import tpu_sc as plsc

F = 32
R = 512
HW = R * R
N = 524288
NW = 32
PTS = N // NW
B = 64
NCH = PTS // B
NPAIR = NCH // 2
G = B // 16
NROW = 6 * 128
OUTW = 3 * F


def _axis_decompose(v):
    t = (v - 0.0) / 2.0 + 0.5
    xn = t * 2.0 - 1.0
    pos = (xn + 1.0) * 0.5 * float(R - 1)
    it = pos.astype(jnp.int32)
    itf = it.astype(jnp.float32)
    fl = jnp.where(pos < itf, it - 1, it)
    w = pos - fl.astype(jnp.float32)
    i0 = jnp.clip(fl, 0, R - 1)
    i1 = jnp.clip(fl + 1, 0, R - 1)
    return i0, i1, w


def _body(xt, t0, t1, t2, out,
          xall, yall, zall, wA, wB, idxA, idxB, rowsA, rowsB, outA, outB,
          gsemA, gsemB, osemA, osemB):
    wid = lax.axis_index("s") * 2 + lax.axis_index("c")
    tbase = wid * PTS
    iota2 = lax.iota(jnp.int32, 16) * 2
    tbls = (t0, t0, t1, t1, t2, t2)

    pltpu.sync_copy(xt.at[0, pl.ds(tbase, PTS)], xall)
    pltpu.sync_copy(xt.at[1, pl.ds(tbase, PTS)], yall)
    pltpu.sync_copy(xt.at[2, pl.ds(tbase, PTS)], zall)

    def build(kc, idx_t, w_t):
        cb = kc * B

        def grp(g, c2):
            col = g * 16
            sl = pl.ds(cb + col, 16)
            x0, x1, wx = _axis_decompose(xall[sl])
            y0, y1, wy = _axis_decompose(yall[sl])
            z0, z1, wz = _axis_decompose(zall[sl])
            w_t[pl.ds(col, 16)] = wx
            w_t[pl.ds(B + col, 16)] = wy
            w_t[pl.ds(2 * B + col, 16)] = wz
            planes = ((x0, x1, y0, y1), (x0, x1, z0, z1), (y0, y1, z0, z1))
            for p, (a0, a1, b0, b1) in enumerate(planes):
                rb0 = b0 * R
                rb1 = b1 * R
                for c, rr in enumerate((rb0 + a0, rb0 + a1, rb1 + a0, rb1 + a1)):
                    q = 4 * p + c
                    idx_t[q // 2, pl.ds((q % 2) * 64 + col, 16)] = rr
            return c2

        lax.fori_loop(0, G, grp, 0)

    def fire_gather(idx_t, rows_t, sem):
        for j in range(6):
            pltpu.make_async_copy(tbls[j].at[idx_t.at[j]],
                                  rows_t.at[pl.ds(j * 128, 128)], sem).start()

    def wait_gather(idx_t, rows_t, sem):
        for j in range(6):
            pltpu.make_async_copy(tbls[j].at[idx_t.at[j]],
                                  rows_t.at[pl.ds(j * 128, 128)], sem).wait()

    def combine(rows_t, w_t, out_t):
        def cgrp(g, c2):
            col = g * 16
            wxg = w_t[pl.ds(col, 16)]
            wyg = w_t[pl.ds(B + col, 16)]
            wzg = w_t[pl.ds(2 * B + col, 16)]
            for j in range(16):
                i = col + j
                jv = jnp.full((16,), j, jnp.int32)
                fmt = plsc.PackFormat.INTERLEAVED
                wxv = wxg.at[jv].get(mode="promise_in_bounds")
                wyv = wyg.at[jv].get(mode="promise_in_bounds")
                wzv = wzg.at[jv].get(mode="promise_in_bounds")
                wxb = plsc.pack(wxv, wxv, format=fmt)
                wyb = plsc.pack(wyv, wyv, format=fmt)
                wzb = plsc.pack(wzv, wzv, format=fmt)
                obase = i * OUTW
                pw = ((wxb, wyb), (wxb, wzb), (wyb, wzb))
                for p, (wa, wb) in enumerate(pw):
                    v00 = rows_t[(4 * p + 0) * 64 + i, :]
                    v01 = rows_t[(4 * p + 1) * 64 + i, :]
                    v10 = rows_t[(4 * p + 2) * 64 + i, :]
                    v11 = rows_t[(4 * p + 3) * 64 + i, :]
                    top = v00 + wa * (v01 - v00)
                    bot = v10 + wa * (v11 - v10)
                    res = top + wb * (bot - top)
                    ue, uo = plsc.unpack(res, format=fmt)
                    out_t[pl.ds(obase + p * F, 16)] = ue
                    out_t[pl.ds(obase + p * F + 16, 16)] = uo
            return c2

        lax.fori_loop(0, G, cgrp, 0)

    def out_desc(kc, out_t, sem):
        off = (tbase + kc * B) * OUTW
        return pltpu.make_async_copy(out_t, out.at[pl.ds(off, B * OUTW)], sem)

    build(0, idxA, wA)
    fire_gather(idxA, rowsA, gsemA)

    def pair(j, carry):
        k0 = j * 2

        build(k0 + 1, idxB, wB)
        fire_gather(idxB, rowsB, gsemB)
        wait_gather(idxA, rowsA, gsemA)

        @pl.when(j > 0)
        def _():
            out_desc(k0 - 2, outA, osemA).wait()

        combine(rowsA, wA, outA)
        out_desc(k0, outA, osemA).start()

        @pl.when(j < NPAIR - 1)
        def _():
            build(k0 + 2, idxA, wA)
            fire_gather(idxA, rowsA, gsemA)

        wait_gather(idxB, rowsB, gsemB)

        @pl.when(j > 0)
        def _():
            out_desc(k0 - 1, outB, osemB).wait()

        combine(rowsB, wB, outB)
        out_desc(k0 + 1, outB, osemB).start()
        return carry

    lax.fori_loop(0, NPAIR, pair, 0)

    out_desc(NCH - 2, outA, osemA).wait()
    out_desc(NCH - 1, outB, osemB).wait()


_tri = pl.kernel(
    _body,
    out_type=jax.ShapeDtypeStruct((N * OUTW,), jnp.float32),
    mesh=plsc.VectorSubcoreMesh(core_axis_name="c", subcore_axis_name="s"),
    compiler_params=pltpu.CompilerParams(use_tc_tiling_on_sc=False,
                                         needs_layout_passes=False,
                                         disable_bounds_checks=True,
                                         disable_semaphore_checks=True),
    scratch_types=[
        pltpu.VMEM((PTS,), jnp.float32),
        pltpu.VMEM((PTS,), jnp.float32),
        pltpu.VMEM((PTS,), jnp.float32),
        pltpu.VMEM((3 * B,), jnp.float32),
        pltpu.VMEM((3 * B,), jnp.float32),
        pltpu.VMEM((6, 128), jnp.int32),
        pltpu.VMEM((6, 128), jnp.int32),
        pltpu.VMEM((NROW, F), jnp.bfloat16),
        pltpu.VMEM((NROW, F), jnp.bfloat16),
        pltpu.VMEM((B * OUTW,), jnp.float32),
        pltpu.VMEM((B * OUTW,), jnp.float32),
        pltpu.SemaphoreType.DMA,
        pltpu.SemaphoreType.DMA,
        pltpu.SemaphoreType.DMA,
        pltpu.SemaphoreType.DMA,
    ],
)


_FPERM = jnp.array(
    [v for k in range(F // 2) for v in (k, F // 2 + k)], dtype=jnp.int32)


def _hwc_table(plane):
    return plane[0].transpose(1, 2, 0)[:, :, _FPERM].reshape(HW, F).astype(jnp.bfloat16)


@jax.jit
def kernel(x, plane_xy, plane_xz, plane_yz):
    flat = _tri(x.T,
                _hwc_table(plane_xy), _hwc_table(plane_xz), _hwc_table(plane_yz))
    return flat.reshape(N, OUTW)
